# baseline (speedup 1.0000x reference)
import functools

import jax
import jax.numpy as jnp
import numpy as np
from jax import lax
from jax.experimental import pallas as pl
from jax.experimental.pallas import tpu as pltpu
from jax.experimental.pallas import tpu_sc as plsc

ITEM_NUM = 100000
EMBED_DIM = 64
INTERE_NUM = 4
SAMPLE_NUM = 1280
BATCH = 1024
SEQ_LEN = 20
N_PAIR = (SEQ_LEN - 1) * BATCH

_INFO = plsc.get_sparse_core_info()
_NC, _NS = _INFO.num_cores, _INFO.num_subcores
_NW = _NC * _NS
_PPW = N_PAIR // _NW
_CHUNK = 128
_NCHUNK = SAMPLE_NUM // _CHUNK
_NGRP = SAMPLE_NUM // 16
_NWORD = EMBED_DIM // 2
_NWORD8 = EMBED_DIM // 4

_SC_PARAMS = pltpu.CompilerParams(
    use_tc_tiling_on_sc=False, needs_layout_passes=False
)


def _make_seq_gather(n_rows, d):
    per_w = n_rows // _NW
    chunks = per_w // _CHUNK
    mesh = plsc.VectorSubcoreMesh(core_axis_name="c", subcore_axis_name="s")

    @functools.partial(
        pl.kernel,
        out_type=jax.ShapeDtypeStruct((n_rows, d), jnp.float32),
        mesh=mesh,
        compiler_params=_SC_PARAMS,
        scratch_types=[
            pltpu.VMEM((chunks, _CHUNK), jnp.int32),
            pltpu.VMEM((_CHUNK, d), jnp.float32),
            pltpu.SemaphoreType.DMA,
        ],
    )
    def k(table_hbm, idx_hbm, out_hbm, idx_v, rows_v, sem):
        wid = lax.axis_index("s") * _NC + lax.axis_index("c")
        base = wid * per_w
        for j in range(chunks):
            pltpu.sync_copy(idx_hbm.at[pl.ds(base + j * _CHUNK, _CHUNK)], idx_v.at[j])
        for j in range(chunks):
            pltpu.async_copy(table_hbm.at[idx_v.at[j]], rows_v, sem).wait()
            pltpu.sync_copy(rows_v, out_hbm.at[pl.ds(base + j * _CHUNK, _CHUNK)])

    return k


def _normalize_table_kernel(x_ref, o_ref):
    x = x_ref[...]
    n2 = jnp.sum(x * x, axis=1, keepdims=True)
    y = x * lax.rsqrt(n2) * 127.0
    y = y + jnp.where(y >= 0.0, 0.5, -0.5)
    o_ref[...] = y.astype(jnp.int8)


def _normalize_table(table):
    blk = 2000
    return pl.pallas_call(
        _normalize_table_kernel,
        grid=(ITEM_NUM // blk,),
        in_specs=[pl.BlockSpec((blk, EMBED_DIM), lambda i: (i, 0))],
        out_specs=pl.BlockSpec((blk, EMBED_DIM), lambda i: (i, 0)),
        out_shape=jax.ShapeDtypeStruct((ITEM_NUM, EMBED_DIM), jnp.int8),
    )(table)


def _seg_weights():
    w_all = np.zeros((SEQ_LEN - 1, SEQ_LEN, INTERE_NUM), np.float64)
    for i in range(1, SEQ_LEN):
        seg = (np.arange(i) * INTERE_NUM) // i
        oh = np.eye(INTERE_NUM)[seg]
        counts = oh.sum(0)
        w = np.zeros((SEQ_LEN, INTERE_NUM))
        w[:i] = oh / np.maximum(counts, 1.0)[None, :]
        for k in range(INTERE_NUM):
            if counts[k] == 0:
                w[:i, k] = 1.0 / i
        w_all[i - 1] = w
    return w_all


_SEG_W = _seg_weights()


def _routing_kernel(e_ref, hit_ref, pos_ref):
    en = []
    for l in range(SEQ_LEN):
        e = e_ref[l]
        n2 = jnp.sum(e * e, axis=1, keepdims=True)
        en.append(e * lax.rsqrt(n2))
    for i in range(1, SEQ_LEN):
        target = en[i]
        vns, ss = [], []
        for k in range(INTERE_NUM):
            vec = None
            for l in range(i):
                w = float(_SEG_W[i - 1, l, k])
                if w == 0.0:
                    continue
                term = en[l] * jnp.float32(w)
                vec = term if vec is None else vec + term
            n2 = jnp.sum(vec * vec, axis=1, keepdims=True)
            vn = vec * lax.rsqrt(n2)
            vns.append(vn)
            ss.append(jnp.sum(vn * target, axis=1, keepdims=True))
        m = jnp.maximum(jnp.maximum(ss[0], ss[1]), jnp.maximum(ss[2], ss[3]))
        hit = vns[3]
        for k in range(2, -1, -1):
            hit = jnp.where(ss[k] == m, vns[k], hit)
        hit_ref[i - 1] = hit
        pos_ref[i - 1] = m[:, 0]


def _routing(e2):
    bb = 128
    return pl.pallas_call(
        _routing_kernel,
        grid=(BATCH // bb,),
        in_specs=[pl.BlockSpec((SEQ_LEN, bb, EMBED_DIM), lambda j: (0, j, 0))],
        out_specs=[
            pl.BlockSpec((SEQ_LEN - 1, bb, EMBED_DIM), lambda j: (0, j, 0)),
            pl.BlockSpec((SEQ_LEN - 1, bb), lambda j: (0, j)),
        ],
        out_shape=[
            jax.ShapeDtypeStruct((SEQ_LEN - 1, BATCH, EMBED_DIM), jnp.float32),
            jax.ShapeDtypeStruct((SEQ_LEN - 1, BATCH), jnp.float32),
        ],
    )(e2)


def _make_neg_kernel():
    mesh = plsc.VectorSubcoreMesh(core_axis_name="c", subcore_axis_name="s")

    @functools.partial(
        pl.kernel,
        out_type=jax.ShapeDtypeStruct((_NW, _PPW, 2, 16), jnp.float32),
        mesh=mesh,
        compiler_params=_SC_PARAMS,
        scratch_types=[
            pltpu.VMEM_SHARED((ITEM_NUM, _NWORD8), jnp.int32),
            pltpu.VMEM((_CHUNK, _NWORD8), jnp.int32),
            pltpu.VMEM((_CHUNK, _NWORD8), jnp.int32),
            pltpu.VMEM((_NCHUNK, _CHUNK), jnp.int32),
            pltpu.VMEM((_NCHUNK, _CHUNK), jnp.int32),
            pltpu.VMEM((_NWORD, 16), jnp.int32),
            pltpu.VMEM((_NWORD, 16), jnp.int32),
            pltpu.VMEM((SAMPLE_NUM,), jnp.float32),
            pltpu.VMEM((2, 16), jnp.float32),
            pltpu.VMEM((2, 16), jnp.float32),
            pltpu.SemaphoreType.DMA,
            pltpu.SemaphoreType.DMA,
            pltpu.SemaphoreType.DMA,
            pltpu.SemaphoreType.DMA,
        ],
    )
    def k(tbl_hbm, nidx_hbm, hpk_hbm, out_hbm, spm, rows0, rows1, idx0, idx1,
          h0, h1, scores, ms0, ms1, semr0, semr1, semh, semo):
        rows = (rows0, rows1)
        idxb = (idx0, idx1)
        hb = (h0, h1)
        msb = (ms0, ms1)
        semr = (semr0, semr1)
        sid = lax.axis_index("s")
        wid = sid * _NC + lax.axis_index("c")
        p0 = wid * _PPW

        @pl.when(sid == 0)
        def _():
            pltpu.sync_copy(tbl_hbm, spm)

        plsc.subcore_barrier()

        def fire_chunk(par, ibuf, c):
            src = spm if par == 0 else tbl_hbm
            for kk in range(_CHUNK // 16):
                iv = idxb[ibuf][c, pl.ds(kk * 16, 16)]
                pltpu.async_copy(
                    src.at[iv], rows[par].at[pl.ds(kk * 16, 16)], semr[par]
                )

        def drain_chunk(par):
            src = spm if par == 0 else tbl_hbm
            pltpu.make_async_copy(
                src.at[pl.ds(0, _CHUNK)], rows[par], semr[par]
            ).wait()

        def stage_async(buf, pair):
            pltpu.async_copy(nidx_hbm.at[pair], idxb[buf], semh)
            pltpu.async_copy(hpk_hbm.at[pair], hb[buf], semh)

        def drain_stage():
            pltpu.make_async_copy(nidx_hbm.at[0], idxb[0], semh).wait()
            pltpu.make_async_copy(hpk_hbm.at[0], hb[0], semh).wait()

        pltpu.sync_copy(nidx_hbm.at[p0], idxb[0])
        pltpu.sync_copy(hpk_hbm.at[p0], hb[0])
        fire_chunk(0, 0, 0)
        stage_async(1, p0 + 1)

        riota = jnp.arange(16, dtype=jnp.int32)

        def body(pp, carry):
            for b in range(2):
                p_local = pp * 2 + b
                p = p0 + p_local
                pbuf = b

                @pl.when(p_local + 1 < _PPW)
                def _():
                    drain_stage()

                hreg = [
                    plsc.bitcast(hb[pbuf][w], jnp.bfloat16) for w in range(_NWORD)
                ]

                def grp(par, coff, g, m_vec):
                    ridx = riota + g * 16
                    acc = jnp.zeros((2 * 16,), jnp.bfloat16)
                    for w in range(_NWORD8):
                        x = plsc.load_gather(
                            rows[par], [ridx, jnp.full((16,), w, jnp.int32)]
                        )
                        b0 = ((x << 24) >> 24).astype(jnp.float32)
                        b1 = ((x << 16) >> 24).astype(jnp.float32)
                        b2 = ((x << 8) >> 24).astype(jnp.float32)
                        b3 = (x >> 24).astype(jnp.float32)
                        p01 = plsc.pack(b0, b1, format=plsc.PackFormat.INTERLEAVED)
                        p23 = plsc.pack(b2, b3, format=plsc.PackFormat.INTERLEAVED)
                        acc = acc + p01 * hreg[2 * w] + p23 * hreg[2 * w + 1]
                    a, bb = plsc.unpack(acc, format=plsc.PackFormat.INTERLEAVED)
                    sc = a + bb
                    scores[pl.ds(coff + g * 16, 16)] = sc
                    return jnp.maximum(m_vec, sc)

                def chunks(cc, m_vec):
                    fire_chunk(1, pbuf, 2 * cc + 1)
                    drain_chunk(0)
                    coff0 = (2 * cc) * _CHUNK
                    m_vec = lax.fori_loop(
                        0, _CHUNK // 16,
                        lambda g, mv: grp(0, coff0, g, mv), m_vec)

                    @pl.when(cc < _NCHUNK // 2 - 1)
                    def _():
                        fire_chunk(0, pbuf, 2 * cc + 2)

                    @pl.when((cc == _NCHUNK // 2 - 1) & (p_local + 1 < _PPW))
                    def _():
                        fire_chunk(0, 1 - pbuf, 0)

                    drain_chunk(1)
                    coff1 = (2 * cc + 1) * _CHUNK
                    m_vec = lax.fori_loop(
                        0, _CHUNK // 16,
                        lambda g, mv: grp(1, coff1, g, mv), m_vec)
                    return m_vec

                m_vec = lax.fori_loop(
                    0, _NCHUNK // 2, chunks,
                    jnp.full((16,), -jnp.inf, jnp.float32))

                def grp2(g, s_vec):
                    x = scores[pl.ds(g * 16, 16)]
                    return s_vec + jnp.exp(x - m_vec)

                s_vec = lax.fori_loop(
                    0, _NGRP, grp2, jnp.zeros((16,), jnp.float32))

                @pl.when(p_local >= 2)
                def _():
                    pltpu.make_async_copy(
                        msb[pbuf], out_hbm.at[wid, 0], semo).wait()

                msb[pbuf][0] = m_vec
                msb[pbuf][1] = s_vec
                pltpu.async_copy(msb[pbuf], out_hbm.at[wid, p_local], semo)

                @pl.when(p_local + 2 < _PPW)
                def _():
                    stage_async(pbuf, p + 2)

            return carry

        lax.fori_loop(0, _PPW // 2, body, jnp.int32(0))
        pltpu.make_async_copy(msb[0], out_hbm.at[wid, 0], semo).wait()
        pltpu.make_async_copy(msb[1], out_hbm.at[wid, 0], semo).wait()

    return k


def _finalize_kernel(ms_ref, pos_ref, o_ref):
    ms = ms_ref[...]
    mv = ms[:, :16]
    sv = ms[:, 16:]
    big_m = jnp.max(mv, axis=1, keepdims=True)
    big_s = jnp.sum(sv * jnp.exp(mv - big_m), axis=1)
    lse = big_m[:, 0] + jnp.log(big_s)
    o_ref[...] = jnp.reshape(jnp.sum(lse) - jnp.sum(pos_ref[...]), (1, 1))


def _finalize(ms, pos):
    return pl.pallas_call(
        _finalize_kernel,
        out_shape=jax.ShapeDtypeStruct((1, 1), jnp.float32),
    )(ms, pos)


def kernel(seqs, item_table, neg_idx):
    seqs = seqs.astype(jnp.int32)
    neg_idx = neg_idx.astype(jnp.int32)

    flat_idx = seqs.T.reshape(-1)
    gathered = _make_seq_gather(BATCH * SEQ_LEN, EMBED_DIM)(item_table, flat_idx)
    e2 = gathered.reshape(SEQ_LEN, BATCH, EMBED_DIM)

    hitted, pos = _routing(e2)

    tbl_i8 = _normalize_table(item_table)
    tbl_i32 = lax.bitcast_convert_type(
        tbl_i8.reshape(ITEM_NUM, _NWORD8, 4), jnp.int32
    )

    hb = (hitted * jnp.float32(1.0 / 127.0)).astype(jnp.bfloat16)
    hb = hb.reshape(N_PAIR, _NWORD, 1, 2)
    hpk = lax.bitcast_convert_type(
        jnp.broadcast_to(hb, (N_PAIR, _NWORD, 16, 2)), jnp.int32
    )

    nidx3 = neg_idx.reshape(N_PAIR, _NCHUNK, _CHUNK)

    ms = _make_neg_kernel()(tbl_i32, nidx3, hpk)
    ms_flat = ms.reshape(N_PAIR, 32)

    out = _finalize(ms_flat, pos.reshape(152, 128))
    return out[0, 0]


# speedup vs baseline: 1.0546x; 1.0546x over previous
import functools

import jax
import jax.numpy as jnp
import numpy as np
from jax import lax
from jax.experimental import pallas as pl
from jax.experimental.pallas import tpu as pltpu
from jax.experimental.pallas import tpu_sc as plsc

ITEM_NUM = 100000
EMBED_DIM = 64
INTERE_NUM = 4
SAMPLE_NUM = 1280
BATCH = 1024
SEQ_LEN = 20
N_PAIR = (SEQ_LEN - 1) * BATCH

_INFO = plsc.get_sparse_core_info()
_NC, _NS = _INFO.num_cores, _INFO.num_subcores
_NW = _NC * _NS
_PPW = N_PAIR // _NW
_CHUNK = 128
_NCHUNK = SAMPLE_NUM // _CHUNK
_NGRP = SAMPLE_NUM // 16
_NWORD = EMBED_DIM // 2
_NWORD8 = EMBED_DIM // 4

_SC_PARAMS = pltpu.CompilerParams(
    use_tc_tiling_on_sc=False, needs_layout_passes=False
)


def _make_seq_gather(n_rows, d):
    per_w = n_rows // _NW
    chunks = per_w // _CHUNK
    mesh = plsc.VectorSubcoreMesh(core_axis_name="c", subcore_axis_name="s")

    @functools.partial(
        pl.kernel,
        out_type=jax.ShapeDtypeStruct((n_rows, d), jnp.float32),
        mesh=mesh,
        compiler_params=_SC_PARAMS,
        scratch_types=[
            pltpu.VMEM((chunks, _CHUNK), jnp.int32),
            pltpu.VMEM((_CHUNK, d), jnp.float32),
            pltpu.SemaphoreType.DMA,
        ],
    )
    def k(table_hbm, idx_hbm, out_hbm, idx_v, rows_v, sem):
        wid = lax.axis_index("s") * _NC + lax.axis_index("c")
        base = wid * per_w
        for j in range(chunks):
            pltpu.sync_copy(idx_hbm.at[pl.ds(base + j * _CHUNK, _CHUNK)], idx_v.at[j])
        for j in range(chunks):
            pltpu.async_copy(table_hbm.at[idx_v.at[j]], rows_v, sem).wait()
            pltpu.sync_copy(rows_v, out_hbm.at[pl.ds(base + j * _CHUNK, _CHUNK)])

    return k


def _normalize_table_kernel(x_ref, o_ref):
    x = x_ref[...]
    n2 = jnp.sum(x * x, axis=1, keepdims=True)
    y = x * lax.rsqrt(n2) * 127.0
    y = y + jnp.where(y >= 0.0, 0.5, -0.5)
    o_ref[...] = y.astype(jnp.int8)


def _normalize_table(table):
    blk = 2000
    return pl.pallas_call(
        _normalize_table_kernel,
        grid=(ITEM_NUM // blk,),
        in_specs=[pl.BlockSpec((blk, EMBED_DIM), lambda i: (i, 0))],
        out_specs=pl.BlockSpec((blk, EMBED_DIM), lambda i: (i, 0)),
        out_shape=jax.ShapeDtypeStruct((ITEM_NUM, EMBED_DIM), jnp.int8),
    )(table)


def _seg_weights():
    w_all = np.zeros((SEQ_LEN - 1, SEQ_LEN, INTERE_NUM), np.float64)
    for i in range(1, SEQ_LEN):
        seg = (np.arange(i) * INTERE_NUM) // i
        oh = np.eye(INTERE_NUM)[seg]
        counts = oh.sum(0)
        w = np.zeros((SEQ_LEN, INTERE_NUM))
        w[:i] = oh / np.maximum(counts, 1.0)[None, :]
        for k in range(INTERE_NUM):
            if counts[k] == 0:
                w[:i, k] = 1.0 / i
        w_all[i - 1] = w
    return w_all


_SEG_W = _seg_weights()


def _routing_kernel(e_ref, hit_ref, pos_ref):
    en = []
    for l in range(SEQ_LEN):
        e = e_ref[l]
        n2 = jnp.sum(e * e, axis=1, keepdims=True)
        en.append(e * lax.rsqrt(n2))
    for i in range(1, SEQ_LEN):
        target = en[i]
        vns, ss = [], []
        for k in range(INTERE_NUM):
            vec = None
            for l in range(i):
                w = float(_SEG_W[i - 1, l, k])
                if w == 0.0:
                    continue
                term = en[l] * jnp.float32(w)
                vec = term if vec is None else vec + term
            n2 = jnp.sum(vec * vec, axis=1, keepdims=True)
            vn = vec * lax.rsqrt(n2)
            vns.append(vn)
            ss.append(jnp.sum(vn * target, axis=1, keepdims=True))
        m = jnp.maximum(jnp.maximum(ss[0], ss[1]), jnp.maximum(ss[2], ss[3]))
        hit = vns[3]
        for k in range(2, -1, -1):
            hit = jnp.where(ss[k] == m, vns[k], hit)
        hit_ref[i - 1] = hit
        pos_ref[i - 1] = m[:, 0]


def _routing(e2):
    bb = 128
    return pl.pallas_call(
        _routing_kernel,
        grid=(BATCH // bb,),
        in_specs=[pl.BlockSpec((SEQ_LEN, bb, EMBED_DIM), lambda j: (0, j, 0))],
        out_specs=[
            pl.BlockSpec((SEQ_LEN - 1, bb, EMBED_DIM), lambda j: (0, j, 0)),
            pl.BlockSpec((SEQ_LEN - 1, bb), lambda j: (0, j)),
        ],
        out_shape=[
            jax.ShapeDtypeStruct((SEQ_LEN - 1, BATCH, EMBED_DIM), jnp.float32),
            jax.ShapeDtypeStruct((SEQ_LEN - 1, BATCH), jnp.float32),
        ],
    )(e2)


def _make_neg_kernel():
    mesh = plsc.VectorSubcoreMesh(core_axis_name="c", subcore_axis_name="s")

    @functools.partial(
        pl.kernel,
        out_type=jax.ShapeDtypeStruct((_NW, _PPW, 2, 16), jnp.float32),
        mesh=mesh,
        compiler_params=_SC_PARAMS,
        scratch_types=[
            pltpu.VMEM_SHARED((ITEM_NUM, _NWORD8), jnp.int32),
            pltpu.VMEM((_CHUNK, _NWORD8), jnp.int32),
            pltpu.VMEM((_CHUNK, _NWORD8), jnp.int32),
            pltpu.VMEM((_NCHUNK, _CHUNK), jnp.int32),
            pltpu.VMEM((_NCHUNK, _CHUNK), jnp.int32),
            pltpu.VMEM((_NWORD, 16), jnp.int32),
            pltpu.VMEM((_NWORD, 16), jnp.int32),
            pltpu.VMEM((SAMPLE_NUM,), jnp.float32),
            pltpu.VMEM((2, 16), jnp.float32),
            pltpu.VMEM((2, 16), jnp.float32),
            pltpu.SemaphoreType.DMA,
            pltpu.SemaphoreType.DMA,
            pltpu.SemaphoreType.DMA,
            pltpu.SemaphoreType.DMA,
        ],
    )
    def k(tbl_hbm, nidx_hbm, hpk_hbm, out_hbm, spm, rows0, rows1, idx0, idx1,
          h0, h1, scores, ms0, ms1, semr0, semr1, semh, semo):
        rows = (rows0, rows1)
        idxb = (idx0, idx1)
        hb = (h0, h1)
        msb = (ms0, ms1)
        semr = (semr0, semr1)
        sid = lax.axis_index("s")
        wid = sid * _NC + lax.axis_index("c")
        p0 = wid * _PPW

        @pl.when(sid == 0)
        def _():
            pltpu.sync_copy(tbl_hbm, spm)

        plsc.subcore_barrier()

        def fire_chunk(par, ibuf, c):
            src = spm if par == 0 else tbl_hbm
            pltpu.async_copy(src.at[idxb[ibuf].at[c]], rows[par], semr[par])

        def drain_chunk(par):
            src = spm if par == 0 else tbl_hbm
            pltpu.make_async_copy(
                src.at[pl.ds(0, _CHUNK)], rows[par], semr[par]
            ).wait()

        def stage_async(buf, pair):
            pltpu.async_copy(nidx_hbm.at[pair], idxb[buf], semh)
            pltpu.async_copy(hpk_hbm.at[pair], hb[buf], semh)

        def drain_stage():
            pltpu.make_async_copy(nidx_hbm.at[0], idxb[0], semh).wait()
            pltpu.make_async_copy(hpk_hbm.at[0], hb[0], semh).wait()

        pltpu.sync_copy(nidx_hbm.at[p0], idxb[0])
        pltpu.sync_copy(hpk_hbm.at[p0], hb[0])
        fire_chunk(0, 0, 0)
        stage_async(1, p0 + 1)

        riota = jnp.arange(16, dtype=jnp.int32)

        def body(pp, carry):
            for b in range(2):
                p_local = pp * 2 + b
                p = p0 + p_local
                pbuf = b

                @pl.when(p_local + 1 < _PPW)
                def _():
                    drain_stage()

                hreg = [
                    plsc.bitcast(hb[pbuf][w], jnp.bfloat16) for w in range(_NWORD)
                ]

                def grp(par, coff, g, m_vec):
                    ridx = riota + g * 16
                    acc = jnp.zeros((2 * 16,), jnp.bfloat16)
                    for w in range(_NWORD8):
                        x = plsc.load_gather(
                            rows[par], [ridx, jnp.full((16,), w, jnp.int32)]
                        )
                        b0 = ((x << 24) >> 24).astype(jnp.float32)
                        b1 = ((x << 16) >> 24).astype(jnp.float32)
                        b2 = ((x << 8) >> 24).astype(jnp.float32)
                        b3 = (x >> 24).astype(jnp.float32)
                        p01 = plsc.pack(b0, b1, format=plsc.PackFormat.INTERLEAVED)
                        p23 = plsc.pack(b2, b3, format=plsc.PackFormat.INTERLEAVED)
                        acc = acc + p01 * hreg[2 * w] + p23 * hreg[2 * w + 1]
                    a, bb = plsc.unpack(acc, format=plsc.PackFormat.INTERLEAVED)
                    sc = a + bb
                    scores[pl.ds(coff + g * 16, 16)] = sc
                    return jnp.maximum(m_vec, sc)

                def chunks(cc, m_vec):
                    fire_chunk(1, pbuf, 2 * cc + 1)
                    drain_chunk(0)
                    coff0 = (2 * cc) * _CHUNK
                    m_vec = lax.fori_loop(
                        0, _CHUNK // 16,
                        lambda g, mv: grp(0, coff0, g, mv), m_vec)

                    @pl.when(cc < _NCHUNK // 2 - 1)
                    def _():
                        fire_chunk(0, pbuf, 2 * cc + 2)

                    @pl.when((cc == _NCHUNK // 2 - 1) & (p_local + 1 < _PPW))
                    def _():
                        fire_chunk(0, 1 - pbuf, 0)

                    drain_chunk(1)
                    coff1 = (2 * cc + 1) * _CHUNK
                    m_vec = lax.fori_loop(
                        0, _CHUNK // 16,
                        lambda g, mv: grp(1, coff1, g, mv), m_vec)
                    return m_vec

                m_vec = lax.fori_loop(
                    0, _NCHUNK // 2, chunks,
                    jnp.full((16,), -jnp.inf, jnp.float32))

                def grp2(g, s_vec):
                    x = scores[pl.ds(g * 16, 16)]
                    return s_vec + jnp.exp(x - m_vec)

                s_vec = lax.fori_loop(
                    0, _NGRP, grp2, jnp.zeros((16,), jnp.float32))

                @pl.when(p_local >= 2)
                def _():
                    pltpu.make_async_copy(
                        msb[pbuf], out_hbm.at[wid, 0], semo).wait()

                msb[pbuf][0] = m_vec
                msb[pbuf][1] = s_vec
                pltpu.async_copy(msb[pbuf], out_hbm.at[wid, p_local], semo)

                @pl.when(p_local + 2 < _PPW)
                def _():
                    stage_async(pbuf, p + 2)

            return carry

        lax.fori_loop(0, _PPW // 2, body, jnp.int32(0))
        pltpu.make_async_copy(msb[0], out_hbm.at[wid, 0], semo).wait()
        pltpu.make_async_copy(msb[1], out_hbm.at[wid, 0], semo).wait()

    return k


def _finalize_kernel(ms_ref, pos_ref, o_ref):
    ms = ms_ref[...]
    mv = ms[:, :16]
    sv = ms[:, 16:]
    big_m = jnp.max(mv, axis=1, keepdims=True)
    big_s = jnp.sum(sv * jnp.exp(mv - big_m), axis=1)
    lse = big_m[:, 0] + jnp.log(big_s)
    o_ref[...] = jnp.reshape(jnp.sum(lse) - jnp.sum(pos_ref[...]), (1, 1))


def _finalize(ms, pos):
    return pl.pallas_call(
        _finalize_kernel,
        out_shape=jax.ShapeDtypeStruct((1, 1), jnp.float32),
    )(ms, pos)


def kernel(seqs, item_table, neg_idx):
    seqs = seqs.astype(jnp.int32)
    neg_idx = neg_idx.astype(jnp.int32)

    flat_idx = seqs.T.reshape(-1)
    gathered = _make_seq_gather(BATCH * SEQ_LEN, EMBED_DIM)(item_table, flat_idx)
    e2 = gathered.reshape(SEQ_LEN, BATCH, EMBED_DIM)

    hitted, pos = _routing(e2)

    tbl_i8 = _normalize_table(item_table)
    tbl_i32 = lax.bitcast_convert_type(
        tbl_i8.reshape(ITEM_NUM, _NWORD8, 4), jnp.int32
    )

    hb = (hitted * jnp.float32(1.0 / 127.0)).astype(jnp.bfloat16)
    hb = hb.reshape(N_PAIR, _NWORD, 1, 2)
    hpk = lax.bitcast_convert_type(
        jnp.broadcast_to(hb, (N_PAIR, _NWORD, 16, 2)), jnp.int32
    )

    nidx3 = neg_idx.reshape(N_PAIR, _NCHUNK, _CHUNK)

    ms = _make_neg_kernel()(tbl_i32, nidx3, hpk)
    ms_flat = ms.reshape(N_PAIR, 32)

    out = _finalize(ms_flat, pos.reshape(152, 128))
    return out[0, 0]
